# baseline (speedup 1.0000x reference)
import functools

import jax
import jax.numpy as jnp
from jax import lax
from jax.experimental import pallas as pl
from jax.experimental.pallas import tpu as pltpu
from jax.experimental.pallas import tpu_sc as plsc

N = 10000
E = 160000
D = 128
NB = 20
SDIM = 480
NIR = 224
HID = 576
EPS = 1e-5
GW = 512
CG = 128
NG = 5
CH = 128
NCH = E // CH
CH2 = 64
NCH2 = E // CH2
TW2 = 80
TW4 = 80
NTRASH = 8
NW = 32
BN = 1000
BE = 1000


def _m3():
    c = lax.broadcasted_iota(jnp.int32, (64, 192), 0)
    r = lax.broadcasted_iota(jnp.int32, (64, 192), 1)
    return (r // 3 == c).astype(jnp.float32)


def _m5():
    c = lax.broadcasted_iota(jnp.int32, (32, 160), 0)
    r = lax.broadcasted_iota(jnp.int32, (32, 160), 1)
    return (r // 5 == c).astype(jnp.float32)


def _pack_bf16_pair(a, b):
    ua = lax.bitcast_convert_type(a, jnp.uint32)
    ub = lax.bitcast_convert_type(b, jnp.uint32)
    ra = (ua + jnp.uint32(0x7FFF) + ((ua >> 16) & jnp.uint32(1))) >> 16
    rb = (ub + jnp.uint32(0x7FFF) + ((ub >> 16) & jnp.uint32(1))) >> 16
    return lax.bitcast_convert_type(ra | (rb << 16), jnp.int32)


def _unpack_bf16_pair(w):
    lo = lax.bitcast_convert_type(w << 16, jnp.float32)
    hi = lax.bitcast_convert_type(w & jnp.int32(-65536), jnp.float32)
    return lo, hi


def _k1_body(xs_ref, xp_ref, w1_ref, b1_ref, w2_ref, b2_ref, g_ref, b_ref,
             gt_ref, i0_ref, i1_ref, i2_ref, i3_ref, i4_ref):
    xs = xs_ref[...]
    xp = xp_ref[...]
    mu = jnp.mean(xs, axis=-1, keepdims=True)
    xc = xs - mu
    var = jnp.mean(xc * xc, axis=-1, keepdims=True)
    s_in = xc / jnp.sqrt(var + EPS) * g_ref[...] + b_ref[...]
    s = xp[:, :128]
    v = xp[:, 128:320]
    t = xp[:, 320:480]
    s_mu = jnp.mean(s, axis=-1, keepdims=True)
    s_c = s - s_mu
    s_o = s_c / jnp.sqrt(jnp.mean(s_c * s_c, axis=-1, keepdims=True) + EPS)
    v_o = v / jnp.sqrt(3.0 * jnp.mean(v * v, axis=-1, keepdims=True) + EPS)
    t_o = t / jnp.sqrt(5.0 * jnp.mean(t * t, axis=-1, keepdims=True) + EPS)
    h = s_in @ w1_ref[...] + b1_ref[...]
    h = h * jax.nn.sigmoid(h)
    so = h @ w2_ref[...] + b2_ref[...]
    a_s = s_o * so[:, 0:128]
    a_v = v_o * (so[:, 128:192] @ _m3())
    a_t = t_o * (so[:, 192:224] @ _m5())
    z64 = jnp.zeros((xs.shape[0], 64), jnp.float32)
    z96 = jnp.zeros((xs.shape[0], 96), jnp.float32)
    slots = [
        a_s,
        a_v[:, :128],
        jnp.concatenate([a_v[:, 128:192], z64], -1),
        a_t[:, :128],
        jnp.concatenate([a_t[:, 128:160], z96], -1),
        so[:, 224:352],
        jnp.concatenate([so[:, 352:448], z64[:, :32]], -1),
        so[:, 448:576],
    ]
    for k in range(4):
        gt_ref[:, 128 * k:128 * (k + 1)] = _pack_bf16_pair(
            slots[2 * k], slots[2 * k + 1])
    xcat = jnp.concatenate([xp, xs, z64[:, :32]], axis=-1)
    i0_ref[...] = xcat[:, 0:128]
    i1_ref[...] = xcat[:, 128:256]
    i2_ref[...] = xcat[:, 256:384]
    i3_ref[...] = xcat[:, 384:512]
    i4_ref[...] = xcat[:, 512:640]


def _k3_body(g_ref, rbf_ref, fcut_ref, rsh_ref, wr_ref, br_ref,
             o0_ref, o1_ref, o2_ref, o3_ref, o4_ref):
    a_s, a_v0 = _unpack_bf16_pair(g_ref[:, 0:128])
    a_v1, a_t0 = _unpack_bf16_pair(g_ref[:, 128:256])
    a_t1, c_s = _unpack_bf16_pair(g_ref[:, 256:384])
    c_vt, b_ = _unpack_bf16_pair(g_ref[:, 384:512])
    rsh = rsh_ref[...]
    fw = (rbf_ref[...] @ wr_ref[...] + br_ref[...]) * fcut_ref[...]
    m3 = _m3()
    m5 = _m5()
    a_v = jnp.concatenate([a_v0, a_v1[:, :64]], -1)
    a_t = jnp.concatenate([a_t0, a_t1[:, :32]], -1)
    msg_s = a_s * fw[:, 0:128] + rsh[:, 0:128] * c_s * fw[:, 224:352]
    msg_v = a_v * (fw[:, 128:192] @ m3) \
        + rsh[:, 128:320] * ((c_vt[:, 0:64] * fw[:, 352:416]) @ m3)
    msg_t = a_t * (fw[:, 192:224] @ m5) \
        + rsh[:, 320:480] * ((c_vt[:, 64:96] * fw[:, 416:448]) @ m5)
    msg_b = b_ * fw[:, 448:576]
    zpad = jnp.zeros((a_s.shape[0], 32), jnp.float32)
    msg = jnp.concatenate([msg_s, msg_v, msg_t, msg_b, zpad], -1)
    o0_ref[...] = msg[:, 0:128]
    o1_ref[...] = msg[:, 128:256]
    o2_ref[...] = msg[:, 256:384]
    o3_ref[...] = msg[:, 384:512]
    o4_ref[...] = msg[:, 512:640]


def _k5_body(o0_ref, o1_ref, o2_ref, o3_ref, o4_ref, sph_ref, sc_ref):
    sph_ref[...] = jnp.concatenate(
        [o0_ref[...], o1_ref[...], o2_ref[...], o3_ref[:, :96]], -1)
    sc_ref[...] = jnp.concatenate([o3_ref[:, 96:128], o4_ref[:, :96]], -1)


def _make_gather_body(nch, tw):
    def _gather_body(gt, src2d, out, idx_all, buf0, buf1, g0, g1, w0, w1):
        wid = lax.axis_index("s") * 2 + lax.axis_index("c")
        base = wid * tw
        pltpu.sync_copy(src2d.at[pl.ds(base, tw)], idx_all)
        lastk = nch - 1 - base

        def pair(j, carry):
            k0 = j * 2
            k1 = k0 + 1
            ka = jnp.minimum(k0, lastk)
            kb = jnp.minimum(k1, lastk)
            ea = (base + ka) * CH2
            eb = (base + kb) * CH2
            ha = pltpu.async_copy(gt.at[idx_all.at[ka]], buf0, g0)
            hb = pltpu.async_copy(gt.at[idx_all.at[kb]], buf1, g1)
            ha.wait()
            wa = pltpu.async_copy(buf0, out.at[pl.ds(ea, CH2)], w0)
            hb.wait()
            wb = pltpu.async_copy(buf1, out.at[pl.ds(eb, CH2)], w1)
            wa.wait()
            wb.wait()
            return carry

        lax.fori_loop(0, tw // 2, pair, 0)

    return _gather_body


def _make_scatter_body(nch, tw):
    def _scatter_body(m0, m1, m2, m3_, m4, i0, i1, i2, i3, i4, dst2d,
                      o0, o1, o2, o3, o4, idx_all, ib0, ib1, mb0, mb1,
                      acc, ms0, ms1, ss0, ss1):
        cid = lax.axis_index("c")
        sid = lax.axis_index("s")
        base = sid * tw
        lastk = nch - 1 - base
        pltpu.sync_copy(dst2d.at[pl.ds(base, tw)], idx_all)
        lo = cid * (N // 2)

        def remap_into(k, ib):
            for c16 in range(CH // 16):
                v = idx_all[k, pl.ds(c16 * 16, 16)]
                w = v - lo
                ok = (w >= 0) & (w < N // 2)
                ib[0, pl.ds(c16 * 16, 16)] = jnp.where(ok, w, N // 2)

        def run_pass(msg, ini, out, remap, r0, nr, obase):
            pltpu.sync_copy(ini.at[pl.ds(obase + r0, nr)],
                            acc.at[pl.ds(r0, nr)])
            plsc.subcore_barrier()

            def pair(j, carry):
                k0 = j * 2
                k1 = k0 + 1
                ea = (base + jnp.minimum(k0, lastk)) * CH
                eb = (base + jnp.minimum(k1, lastk)) * CH
                ha = pltpu.async_copy(msg.at[pl.ds(ea, CH)], mb0, ms0)
                hb = pltpu.async_copy(msg.at[pl.ds(eb, CH)], mb1, ms1)
                if remap:
                    remap_into(k0, ib0)
                    remap_into(k1, ib1)
                    ia, ib_ = ib0.at[0], ib1.at[0]
                else:
                    ia, ib_ = idx_all.at[k0], idx_all.at[k1]
                ha.wait()
                sa = pltpu.async_copy(mb0, acc.at[ia], ss0, add=True)
                hb.wait()
                sb = pltpu.async_copy(mb1, acc.at[ib_], ss1, add=True)
                sa.wait()
                sb.wait()
                return carry

            lax.fori_loop(0, tw // 2, pair, 0)
            plsc.subcore_barrier()
            pltpu.sync_copy(acc.at[pl.ds(r0, nr)],
                            out.at[pl.ds(obase + r0, nr)])
            plsc.subcore_barrier()

        r0f = sid * 624
        r0h = sid * 312

        @pl.when(cid == 0)
        def _():
            run_pass(m0, i0, o0, False, r0f, 640, 0)
            run_pass(m1, i1, o1, False, r0f, 640, 0)

        @pl.when(cid == 1)
        def _():
            run_pass(m2, i2, o2, False, r0f, 640, 0)
            run_pass(m3_, i3, o3, False, r0f, 640, 0)

        run_pass(m4, i4, o4, True, r0h, 320, lo)

    return _scatter_body


def kernel(x_scalar, x_spherical, rbf, fcut, rsh, edge_index,
           W1, b1, W2, b2, Wr, br, ln_g, ln_b):
    f32 = jnp.float32
    src_i = edge_index[1].astype(jnp.int32)
    dst_i = edge_index[0].astype(jnp.int32)
    nch2 = E // CH2
    tw2 = (nch2 + NW - 1) // NW
    tw2 = tw2 + (tw2 % 2)
    nch4 = E // CH
    tw4 = (nch4 + 15) // 16
    tw4 = tw4 + (tw4 % 2)
    src2d = jnp.pad(src_i, (0, NW * tw2 * CH2 - E)).reshape(NW * tw2, CH2)
    dst2d = jnp.pad(dst_i, (0, 16 * tw4 * CH - E),
                    constant_values=N).reshape(16 * tw4, CH)

    nblk = N // BN
    g_table, i0, i1, i2, i3, i4 = pl.pallas_call(
        _k1_body,
        grid=(nblk,),
        in_specs=[
            pl.BlockSpec((BN, D), lambda i: (i, 0)),
            pl.BlockSpec((BN, SDIM), lambda i: (i, 0)),
            pl.BlockSpec((D, D), lambda i: (0, 0)),
            pl.BlockSpec((1, D), lambda i: (0, 0)),
            pl.BlockSpec((D, HID), lambda i: (0, 0)),
            pl.BlockSpec((1, HID), lambda i: (0, 0)),
            pl.BlockSpec((1, D), lambda i: (0, 0)),
            pl.BlockSpec((1, D), lambda i: (0, 0)),
        ],
        out_specs=[pl.BlockSpec((BN, GW), lambda i: (i, 0))]
        + [pl.BlockSpec((BN, CG), lambda i: (i, 0))] * NG,
        out_shape=[jax.ShapeDtypeStruct((N, GW), jnp.int32)]
        + [jax.ShapeDtypeStruct((N, CG), f32)] * NG,
    )(x_scalar, x_spherical, W1, b1.reshape(1, D), W2, b2.reshape(1, HID),
      ln_g.reshape(1, D), ln_b.reshape(1, D))

    mesh = plsc.VectorSubcoreMesh(core_axis_name="c", subcore_axis_name="s")

    gathered = pl.kernel(
        _make_gather_body(nch2, tw2),
        mesh=mesh,
        out_type=jax.ShapeDtypeStruct((E, GW), jnp.int32),
        scratch_types=[
            pltpu.VMEM((tw2, CH2), jnp.int32),
            pltpu.VMEM((CH2, GW), jnp.int32),
            pltpu.VMEM((CH2, GW), jnp.int32),
            pltpu.SemaphoreType.DMA,
            pltpu.SemaphoreType.DMA,
            pltpu.SemaphoreType.DMA,
            pltpu.SemaphoreType.DMA,
        ],
    )(g_table, src2d)

    m0, m1, m2, m3_, m4 = pl.pallas_call(
        _k3_body,
        grid=(E // BE,),
        in_specs=[
            pl.BlockSpec((BE, GW), lambda i: (i, 0)),
            pl.BlockSpec((BE, NB), lambda i: (i, 0)),
            pl.BlockSpec((BE, 1), lambda i: (i, 0)),
            pl.BlockSpec((BE, SDIM), lambda i: (i, 0)),
            pl.BlockSpec((NB, HID), lambda i: (0, 0)),
            pl.BlockSpec((1, HID), lambda i: (0, 0)),
        ],
        out_specs=[pl.BlockSpec((BE, CG), lambda i: (i, 0))] * NG,
        out_shape=[jax.ShapeDtypeStruct((E, CG), f32)] * NG,
    )(gathered, rbf, fcut, rsh, Wr, br.reshape(1, HID))

    o0, o1, o2, o3, o4 = pl.kernel(
        _make_scatter_body(nch4, tw4),
        mesh=mesh,
        out_type=[jax.ShapeDtypeStruct((N, CG), f32)] * NG,
        scratch_types=[
            pltpu.VMEM((tw4, CH), jnp.int32),
            pltpu.VMEM((1, CH), jnp.int32),
            pltpu.VMEM((1, CH), jnp.int32),
            pltpu.VMEM((CH, CG), f32),
            pltpu.VMEM((CH, CG), f32),
            pltpu.VMEM_SHARED((N + NTRASH, CG), f32),
            pltpu.SemaphoreType.DMA,
            pltpu.SemaphoreType.DMA,
            pltpu.SemaphoreType.DMA,
            pltpu.SemaphoreType.DMA,
        ],
    )(m0, m1, m2, m3_, m4, i0, i1, i2, i3, i4, dst2d)

    new_sph, new_sc = pl.pallas_call(
        _k5_body,
        grid=(nblk,),
        in_specs=[pl.BlockSpec((BN, CG), lambda i: (i, 0))] * NG,
        out_specs=[pl.BlockSpec((BN, SDIM), lambda i: (i, 0)),
                   pl.BlockSpec((BN, D), lambda i: (i, 0))],
        out_shape=[jax.ShapeDtypeStruct((N, SDIM), f32),
                   jax.ShapeDtypeStruct((N, D), f32)],
    )(o0, o1, o2, o3, o4)
    return (new_sc, new_sph)


# speedup vs baseline: 1.0092x; 1.0092x over previous
import jax
import jax.numpy as jnp
from jax import lax
from jax.experimental import pallas as pl
from jax.experimental.pallas import tpu as pltpu
from jax.experimental.pallas import tpu_sc as plsc

N = 10000
E = 160000
D = 128
NB = 20
SDIM = 480
NIR = 224
HID = 576
EPS = 1e-5
GW = 512
CG = 128
NG = 5
CH = 128
NCH = E // CH
CH2 = 64
NCH2 = E // CH2
TW2 = 80
TW4 = 80
NTRASH = 8
NW = 32
BN = 1000
BE = 1000


def _m3():
    c = lax.broadcasted_iota(jnp.int32, (64, 192), 0)
    r = lax.broadcasted_iota(jnp.int32, (64, 192), 1)
    return (r // 3 == c).astype(jnp.float32)


def _m5():
    c = lax.broadcasted_iota(jnp.int32, (32, 160), 0)
    r = lax.broadcasted_iota(jnp.int32, (32, 160), 1)
    return (r // 5 == c).astype(jnp.float32)


def _pack_bf16_pair(a, b):
    ua = lax.bitcast_convert_type(a, jnp.uint32)
    ub = lax.bitcast_convert_type(b, jnp.uint32)
    ra = (ua + jnp.uint32(0x7FFF) + ((ua >> 16) & jnp.uint32(1))) >> 16
    rb = (ub + jnp.uint32(0x7FFF) + ((ub >> 16) & jnp.uint32(1))) >> 16
    return lax.bitcast_convert_type(ra | (rb << 16), jnp.int32)


def _unpack_bf16_pair(w):
    lo = lax.bitcast_convert_type(w << 16, jnp.float32)
    hi = lax.bitcast_convert_type(w & jnp.int32(-65536), jnp.float32)
    return lo, hi


def _k1_body(xs_ref, xp_ref, w1_ref, b1_ref, w2_ref, b2_ref, g_ref, b_ref,
             gt_ref, i0_ref, i1_ref, i2_ref, i3_ref, i4_ref):
    xs = xs_ref[...]
    xp = xp_ref[...]
    mu = jnp.mean(xs, axis=-1, keepdims=True)
    xc = xs - mu
    var = jnp.mean(xc * xc, axis=-1, keepdims=True)
    s_in = xc / jnp.sqrt(var + EPS) * g_ref[...] + b_ref[...]
    s = xp[:, :128]
    v = xp[:, 128:320]
    t = xp[:, 320:480]
    s_mu = jnp.mean(s, axis=-1, keepdims=True)
    s_c = s - s_mu
    s_o = s_c / jnp.sqrt(jnp.mean(s_c * s_c, axis=-1, keepdims=True) + EPS)
    v_o = v / jnp.sqrt(3.0 * jnp.mean(v * v, axis=-1, keepdims=True) + EPS)
    t_o = t / jnp.sqrt(5.0 * jnp.mean(t * t, axis=-1, keepdims=True) + EPS)
    h = s_in @ w1_ref[...] + b1_ref[...]
    h = h * jax.nn.sigmoid(h)
    so = h @ w2_ref[...] + b2_ref[...]
    a_s = s_o * so[:, 0:128]
    a_v = v_o * (so[:, 128:192] @ _m3())
    a_t = t_o * (so[:, 192:224] @ _m5())
    z64 = jnp.zeros((xs.shape[0], 64), jnp.float32)
    z96 = jnp.zeros((xs.shape[0], 96), jnp.float32)
    slots = [
        a_s,
        a_v[:, :128],
        jnp.concatenate([a_v[:, 128:192], z64], -1),
        a_t[:, :128],
        jnp.concatenate([a_t[:, 128:160], z96], -1),
        so[:, 224:352],
        jnp.concatenate([so[:, 352:448], z64[:, :32]], -1),
        so[:, 448:576],
    ]
    for k in range(4):
        gt_ref[:, 128 * k:128 * (k + 1)] = _pack_bf16_pair(
            slots[2 * k], slots[2 * k + 1])
    xcat = jnp.concatenate([xp, xs, z64[:, :32]], axis=-1)
    i0_ref[...] = xcat[:, 0:128]
    i1_ref[...] = xcat[:, 128:256]
    i2_ref[...] = xcat[:, 256:384]
    i3_ref[...] = xcat[:, 384:512]
    i4_ref[...] = xcat[:, 512:640]


def _k3_body(g_ref, rbf_ref, fcut_ref, rsh_ref, wr_ref, br_ref,
             o0_ref, o1_ref, o2_ref, o3_ref, o4_ref):
    a_s, a_v0 = _unpack_bf16_pair(g_ref[:, 0:128])
    a_v1, a_t0 = _unpack_bf16_pair(g_ref[:, 128:256])
    a_t1, c_s = _unpack_bf16_pair(g_ref[:, 256:384])
    c_vt, b_ = _unpack_bf16_pair(g_ref[:, 384:512])
    rsh = rsh_ref[...]
    fw = (rbf_ref[...] @ wr_ref[...] + br_ref[...]) * fcut_ref[...]
    m3 = _m3()
    m5 = _m5()
    a_v = jnp.concatenate([a_v0, a_v1[:, :64]], -1)
    a_t = jnp.concatenate([a_t0, a_t1[:, :32]], -1)
    msg_s = a_s * fw[:, 0:128] + rsh[:, 0:128] * c_s * fw[:, 224:352]
    msg_v = a_v * (fw[:, 128:192] @ m3) \
        + rsh[:, 128:320] * ((c_vt[:, 0:64] * fw[:, 352:416]) @ m3)
    msg_t = a_t * (fw[:, 192:224] @ m5) \
        + rsh[:, 320:480] * ((c_vt[:, 64:96] * fw[:, 416:448]) @ m5)
    msg_b = b_ * fw[:, 448:576]
    zpad = jnp.zeros((a_s.shape[0], 32), jnp.float32)
    o0_ref[...] = msg_s
    o1_ref[...] = msg_v[:, :128]
    o2_ref[...] = jnp.concatenate([msg_v[:, 128:192], msg_t[:, :64]], -1)
    o3_ref[...] = jnp.concatenate([msg_t[:, 64:160], msg_b[:, :32]], -1)
    o4_ref[...] = jnp.concatenate([msg_b[:, 32:128], zpad], -1)


def _k5_body(o0_ref, o1_ref, o2_ref, o3_ref, o4_ref, sph_ref, sc_ref):
    sph_ref[...] = jnp.concatenate(
        [o0_ref[...], o1_ref[...], o2_ref[...], o3_ref[:, :96]], -1)
    sc_ref[...] = jnp.concatenate([o3_ref[:, 96:128], o4_ref[:, :96]], -1)


def _make_gather_body(nch, tw):
    def _gather_body(gt, src2d, out, idx_all, buf0, buf1, g0, g1, w0, w1):
        wid = lax.axis_index("s") * 2 + lax.axis_index("c")
        base = wid * tw
        pltpu.sync_copy(src2d.at[pl.ds(base, tw)], idx_all)
        lastk = nch - 1 - base

        def pair(j, carry):
            k0 = j * 2
            k1 = k0 + 1
            ka = jnp.minimum(k0, lastk)
            kb = jnp.minimum(k1, lastk)
            ea = (base + ka) * CH2
            eb = (base + kb) * CH2
            ha = pltpu.async_copy(gt.at[idx_all.at[ka]], buf0, g0)
            hb = pltpu.async_copy(gt.at[idx_all.at[kb]], buf1, g1)
            ha.wait()
            wa = pltpu.async_copy(buf0, out.at[pl.ds(ea, CH2)], w0)
            hb.wait()
            wb = pltpu.async_copy(buf1, out.at[pl.ds(eb, CH2)], w1)
            wa.wait()
            wb.wait()
            return carry

        lax.fori_loop(0, tw // 2, pair, 0)

    return _gather_body


def _make_scatter_body(nch, tw):
    def _scatter_body(m0, m1, m2, m3_, m4, i0, i1, i2, i3, i4, dst2d,
                      o0, o1, o2, o3, o4, idx_all, mb0, mb1, acc,
                      ms0, ms1, ss0, ss1):
        cid = lax.axis_index("c")
        sid = lax.axis_index("s")
        base = sid * tw
        lastk = nch - 1 - base
        r0 = sid * 624
        nr = 640
        pltpu.sync_copy(dst2d.at[pl.ds(base, tw)], idx_all)

        def one_pass(msg, ini, out):
            pltpu.sync_copy(ini.at[pl.ds(r0, nr)], acc.at[pl.ds(r0, nr)])
            plsc.subcore_barrier()

            def pair(j, carry):
                k0 = j * 2
                k1 = k0 + 1
                ea = (base + jnp.minimum(k0, lastk)) * CH
                eb = (base + jnp.minimum(k1, lastk)) * CH
                ha = pltpu.async_copy(msg.at[pl.ds(ea, CH)], mb0, ms0)
                hb = pltpu.async_copy(msg.at[pl.ds(eb, CH)], mb1, ms1)
                ha.wait()
                sa = pltpu.async_copy(mb0, acc.at[idx_all.at[k0]], ss0,
                                      add=True)
                hb.wait()
                sb = pltpu.async_copy(mb1, acc.at[idx_all.at[k1]], ss1,
                                      add=True)
                sa.wait()
                sb.wait()
                return carry

            lax.fori_loop(0, tw // 2, pair, 0)
            plsc.subcore_barrier()
            pltpu.sync_copy(acc.at[pl.ds(r0, nr)], out.at[pl.ds(r0, nr)])
            plsc.subcore_barrier()

        @pl.when(cid == 0)
        def _():
            one_pass(m0, i0, o0)
            one_pass(m1, i1, o1)

        @pl.when(cid == 1)
        def _():
            one_pass(m2, i2, o2)
            one_pass(m3_, i3, o3)
            one_pass(m4, i4, o4)

    return _scatter_body


def kernel(x_scalar, x_spherical, rbf, fcut, rsh, edge_index,
           W1, b1, W2, b2, Wr, br, ln_g, ln_b):
    f32 = jnp.float32
    src_i = edge_index[1].astype(jnp.int32)
    dst_i = edge_index[0].astype(jnp.int32)
    nch2 = E // CH2
    tw2 = (nch2 + NW - 1) // NW
    tw2 = tw2 + (tw2 % 2)
    nch4 = E // CH
    tw4 = (nch4 + 15) // 16
    tw4 = tw4 + (tw4 % 2)
    src2d = jnp.pad(src_i, (0, NW * tw2 * CH2 - E)).reshape(NW * tw2, CH2)
    dst2d = jnp.pad(dst_i, (0, 16 * tw4 * CH - E),
                    constant_values=N).reshape(16 * tw4, CH)

    nblk = N // BN
    g_table, i0, i1, i2, i3, i4 = pl.pallas_call(
        _k1_body,
        grid=(nblk,),
        in_specs=[
            pl.BlockSpec((BN, D), lambda i: (i, 0)),
            pl.BlockSpec((BN, SDIM), lambda i: (i, 0)),
            pl.BlockSpec((D, D), lambda i: (0, 0)),
            pl.BlockSpec((1, D), lambda i: (0, 0)),
            pl.BlockSpec((D, HID), lambda i: (0, 0)),
            pl.BlockSpec((1, HID), lambda i: (0, 0)),
            pl.BlockSpec((1, D), lambda i: (0, 0)),
            pl.BlockSpec((1, D), lambda i: (0, 0)),
        ],
        out_specs=[pl.BlockSpec((BN, GW), lambda i: (i, 0))]
        + [pl.BlockSpec((BN, CG), lambda i: (i, 0))] * NG,
        out_shape=[jax.ShapeDtypeStruct((N, GW), jnp.int32)]
        + [jax.ShapeDtypeStruct((N, CG), f32)] * NG,
    )(x_scalar, x_spherical, W1, b1.reshape(1, D), W2, b2.reshape(1, HID),
      ln_g.reshape(1, D), ln_b.reshape(1, D))

    mesh = plsc.VectorSubcoreMesh(core_axis_name="c", subcore_axis_name="s")

    gathered = pl.kernel(
        _make_gather_body(nch2, tw2),
        mesh=mesh,
        out_type=jax.ShapeDtypeStruct((E, GW), jnp.int32),
        scratch_types=[
            pltpu.VMEM((tw2, CH2), jnp.int32),
            pltpu.VMEM((CH2, GW), jnp.int32),
            pltpu.VMEM((CH2, GW), jnp.int32),
            pltpu.SemaphoreType.DMA,
            pltpu.SemaphoreType.DMA,
            pltpu.SemaphoreType.DMA,
            pltpu.SemaphoreType.DMA,
        ],
    )(g_table, src2d)

    m0, m1, m2, m3_, m4 = pl.pallas_call(
        _k3_body,
        grid=(E // BE,),
        in_specs=[
            pl.BlockSpec((BE, GW), lambda i: (i, 0)),
            pl.BlockSpec((BE, NB), lambda i: (i, 0)),
            pl.BlockSpec((BE, 1), lambda i: (i, 0)),
            pl.BlockSpec((BE, SDIM), lambda i: (i, 0)),
            pl.BlockSpec((NB, HID), lambda i: (0, 0)),
            pl.BlockSpec((1, HID), lambda i: (0, 0)),
        ],
        out_specs=[pl.BlockSpec((BE, CG), lambda i: (i, 0))] * NG,
        out_shape=[jax.ShapeDtypeStruct((E, CG), f32)] * NG,
    )(gathered, rbf, fcut, rsh, Wr, br.reshape(1, HID))

    o0, o1, o2, o3, o4 = pl.kernel(
        _make_scatter_body(nch4, tw4),
        mesh=mesh,
        out_type=[jax.ShapeDtypeStruct((N, CG), f32)] * NG,
        scratch_types=[
            pltpu.VMEM((tw4, CH), jnp.int32),
            pltpu.VMEM((CH, CG), f32),
            pltpu.VMEM((CH, CG), f32),
            pltpu.VMEM_SHARED((N + NTRASH, CG), f32),
            pltpu.SemaphoreType.DMA,
            pltpu.SemaphoreType.DMA,
            pltpu.SemaphoreType.DMA,
            pltpu.SemaphoreType.DMA,
        ],
    )(m0, m1, m2, m3_, m4, i0, i1, i2, i3, i4, dst2d)

    new_sph, new_sc = pl.pallas_call(
        _k5_body,
        grid=(nblk,),
        in_specs=[pl.BlockSpec((BN, CG), lambda i: (i, 0))] * NG,
        out_specs=[pl.BlockSpec((BN, SDIM), lambda i: (i, 0)),
                   pl.BlockSpec((BN, D), lambda i: (i, 0))],
        out_shape=[jax.ShapeDtypeStruct((N, SDIM), f32),
                   jax.ShapeDtypeStruct((N, D), f32)],
    )(o0, o1, o2, o3, o4)
    return (new_sc, new_sph)


# speedup vs baseline: 1.0488x; 1.0392x over previous
import jax
import jax.numpy as jnp
from jax import lax
from jax.experimental import pallas as pl
from jax.experimental.pallas import tpu as pltpu
from jax.experimental.pallas import tpu_sc as plsc

N = 10000
E = 160000
D = 128
NB = 20
SDIM = 480
NIR = 224
HID = 576
EPS = 1e-5
GW = 512
CG = 128
NG = 5
CH = 128
NCH = E // CH
CH2 = 64
NCH2 = E // CH2
TW2 = 80
TW4 = 80
NTRASH = 8
NW = 32
BN = 1000
BE = 2000


def _m3():
    c = lax.broadcasted_iota(jnp.int32, (64, 192), 0)
    r = lax.broadcasted_iota(jnp.int32, (64, 192), 1)
    return (r // 3 == c).astype(jnp.float32)


def _m5():
    c = lax.broadcasted_iota(jnp.int32, (32, 160), 0)
    r = lax.broadcasted_iota(jnp.int32, (32, 160), 1)
    return (r // 5 == c).astype(jnp.float32)


def _pack_bf16_pair(a, b):
    ua = lax.bitcast_convert_type(a, jnp.uint32)
    ub = lax.bitcast_convert_type(b, jnp.uint32)
    ra = (ua + jnp.uint32(0x7FFF) + ((ua >> 16) & jnp.uint32(1))) >> 16
    rb = (ub + jnp.uint32(0x7FFF) + ((ub >> 16) & jnp.uint32(1))) >> 16
    return lax.bitcast_convert_type(ra | (rb << 16), jnp.int32)


def _unpack_bf16_pair(w):
    lo = lax.bitcast_convert_type(w << 16, jnp.float32)
    hi = lax.bitcast_convert_type(w & jnp.int32(-65536), jnp.float32)
    return lo, hi


def _k1_body(xs_ref, xp_ref, w1_ref, b1_ref, w2_ref, b2_ref, g_ref, b_ref,
             gt_ref, i0_ref, i1_ref, i2_ref, i3_ref, i4_ref):
    xs = xs_ref[...]
    xp = xp_ref[...]
    mu = jnp.mean(xs, axis=-1, keepdims=True)
    xc = xs - mu
    var = jnp.mean(xc * xc, axis=-1, keepdims=True)
    s_in = xc / jnp.sqrt(var + EPS) * g_ref[...] + b_ref[...]
    s = xp[:, :128]
    v = xp[:, 128:320]
    t = xp[:, 320:480]
    s_mu = jnp.mean(s, axis=-1, keepdims=True)
    s_c = s - s_mu
    s_o = s_c / jnp.sqrt(jnp.mean(s_c * s_c, axis=-1, keepdims=True) + EPS)
    v_o = v / jnp.sqrt(3.0 * jnp.mean(v * v, axis=-1, keepdims=True) + EPS)
    t_o = t / jnp.sqrt(5.0 * jnp.mean(t * t, axis=-1, keepdims=True) + EPS)
    h = s_in @ w1_ref[...] + b1_ref[...]
    h = h * jax.nn.sigmoid(h)
    so = h @ w2_ref[...] + b2_ref[...]
    a_s = s_o * so[:, 0:128]
    a_v = v_o * (so[:, 128:192] @ _m3())
    a_t = t_o * (so[:, 192:224] @ _m5())
    z64 = jnp.zeros((xs.shape[0], 64), jnp.float32)
    z96 = jnp.zeros((xs.shape[0], 96), jnp.float32)
    slots = [
        a_s,
        a_v[:, :128],
        jnp.concatenate([a_v[:, 128:192], z64], -1),
        a_t[:, :128],
        jnp.concatenate([a_t[:, 128:160], z96], -1),
        so[:, 224:352],
        jnp.concatenate([so[:, 352:448], z64[:, :32]], -1),
        so[:, 448:576],
    ]
    for k in range(4):
        gt_ref[:, 128 * k:128 * (k + 1)] = _pack_bf16_pair(
            slots[2 * k], slots[2 * k + 1])
    xcat = jnp.concatenate([xp, xs, z64[:, :32]], axis=-1)
    i0_ref[...] = xcat[:, 0:128]
    i1_ref[...] = xcat[:, 128:256]
    i2_ref[...] = xcat[:, 256:384]
    i3_ref[...] = xcat[:, 384:512]
    i4_ref[...] = xcat[:, 512:640]


def _k3_body(g_ref, rbf_ref, fcut_ref, rsh_ref, wr_ref, br_ref,
             o0_ref, o1_ref, o2_ref, o3_ref, o4_ref):
    a_s, a_v0 = _unpack_bf16_pair(g_ref[:, 0:128])
    a_v1, a_t0 = _unpack_bf16_pair(g_ref[:, 128:256])
    a_t1, c_s = _unpack_bf16_pair(g_ref[:, 256:384])
    c_vt, b_ = _unpack_bf16_pair(g_ref[:, 384:512])
    rsh = rsh_ref[...]
    fw = (rbf_ref[...] @ wr_ref[...] + br_ref[...]) * fcut_ref[...]
    m3 = _m3()
    m5 = _m5()
    a_v = jnp.concatenate([a_v0, a_v1[:, :64]], -1)
    a_t = jnp.concatenate([a_t0, a_t1[:, :32]], -1)
    msg_s = a_s * fw[:, 0:128] + rsh[:, 0:128] * c_s * fw[:, 224:352]
    msg_v = a_v * (fw[:, 128:192] @ m3) \
        + rsh[:, 128:320] * ((c_vt[:, 0:64] * fw[:, 352:416]) @ m3)
    msg_t = a_t * (fw[:, 192:224] @ m5) \
        + rsh[:, 320:480] * ((c_vt[:, 64:96] * fw[:, 416:448]) @ m5)
    msg_b = b_ * fw[:, 448:576]
    zpad = jnp.zeros((a_s.shape[0], 32), jnp.float32)
    o0_ref[...] = msg_s
    o1_ref[...] = msg_v[:, :128]
    o2_ref[...] = jnp.concatenate([msg_v[:, 128:192], msg_t[:, :64]], -1)
    o3_ref[...] = jnp.concatenate([msg_t[:, 64:160], msg_b[:, :32]], -1)
    o4_ref[...] = jnp.concatenate([msg_b[:, 32:128], zpad], -1)


def _k5_body(o0_ref, o1_ref, o2_ref, o3_ref, o4_ref, sph_ref, sc_ref):
    sph_ref[...] = jnp.concatenate(
        [o0_ref[...], o1_ref[...], o2_ref[...], o3_ref[:, :96]], -1)
    sc_ref[...] = jnp.concatenate([o3_ref[:, 96:128], o4_ref[:, :96]], -1)


def _make_gather_body(nch, tw):
    def _gather_body(gt, src2d, out, idx_all, buf0, buf1, g0, g1, w0, w1):
        wid = lax.axis_index("s") * 2 + lax.axis_index("c")
        base = wid * tw
        pltpu.sync_copy(src2d.at[pl.ds(base, tw)], idx_all)
        lastk = nch - 1 - base

        def pair(j, carry):
            k0 = j * 2
            k1 = k0 + 1
            ka = jnp.minimum(k0, lastk)
            kb = jnp.minimum(k1, lastk)
            ea = (base + ka) * CH2
            eb = (base + kb) * CH2
            ha = pltpu.async_copy(gt.at[idx_all.at[ka]], buf0, g0)
            hb = pltpu.async_copy(gt.at[idx_all.at[kb]], buf1, g1)
            ha.wait()
            wa = pltpu.async_copy(buf0, out.at[pl.ds(ea, CH2)], w0)
            hb.wait()
            wb = pltpu.async_copy(buf1, out.at[pl.ds(eb, CH2)], w1)
            wa.wait()
            wb.wait()
            return carry

        lax.fori_loop(0, tw // 2, pair, 0)

    return _gather_body


def _make_scatter_body(nch, tw):
    def _scatter_body(m0, m1, m2, m3_, m4, i0, i1, i2, i3, i4, dst2d,
                      o0, o1, o2, o3, o4, idx_all, mb0, mb1, acc,
                      ms0, ms1, ss0, ss1):
        cid = lax.axis_index("c")
        sid = lax.axis_index("s")
        base = sid * tw
        lastk = nch - 1 - base
        r0 = sid * 624
        nr = 640
        pltpu.sync_copy(dst2d.at[pl.ds(base, tw)], idx_all)

        def one_pass(msg, ini, out):
            pltpu.sync_copy(ini.at[pl.ds(r0, nr)], acc.at[pl.ds(r0, nr)])
            plsc.subcore_barrier()

            def pair(j, carry):
                k0 = j * 2
                k1 = k0 + 1
                ea = (base + jnp.minimum(k0, lastk)) * CH
                eb = (base + jnp.minimum(k1, lastk)) * CH
                ha = pltpu.async_copy(msg.at[pl.ds(ea, CH)], mb0, ms0)
                hb = pltpu.async_copy(msg.at[pl.ds(eb, CH)], mb1, ms1)
                ha.wait()
                sa = pltpu.async_copy(mb0, acc.at[idx_all.at[k0]], ss0,
                                      add=True)
                hb.wait()
                sb = pltpu.async_copy(mb1, acc.at[idx_all.at[k1]], ss1,
                                      add=True)
                sa.wait()
                sb.wait()
                return carry

            lax.fori_loop(0, tw // 2, pair, 0)
            plsc.subcore_barrier()
            pltpu.sync_copy(acc.at[pl.ds(r0, nr)], out.at[pl.ds(r0, nr)])
            plsc.subcore_barrier()

        @pl.when(cid == 0)
        def _():
            one_pass(m0, i0, o0)
            one_pass(m1, i1, o1)

        @pl.when(cid == 1)
        def _():
            one_pass(m2, i2, o2)
            one_pass(m3_, i3, o3)
            one_pass(m4, i4, o4)

    return _scatter_body


def kernel(x_scalar, x_spherical, rbf, fcut, rsh, edge_index,
           W1, b1, W2, b2, Wr, br, ln_g, ln_b):
    f32 = jnp.float32
    src_i = edge_index[1].astype(jnp.int32)
    dst_i = edge_index[0].astype(jnp.int32)
    nch2 = E // CH2
    tw2 = (nch2 + NW - 1) // NW
    tw2 = tw2 + (tw2 % 2)
    nch4 = E // CH
    tw4 = (nch4 + 15) // 16
    tw4 = tw4 + (tw4 % 2)
    src2d = jnp.pad(src_i, (0, NW * tw2 * CH2 - E)).reshape(NW * tw2, CH2)
    dst2d = jnp.pad(dst_i, (0, 16 * tw4 * CH - E),
                    constant_values=N).reshape(16 * tw4, CH)

    nblk = N // BN
    g_table, i0, i1, i2, i3, i4 = pl.pallas_call(
        _k1_body,
        grid=(nblk,),
        in_specs=[
            pl.BlockSpec((BN, D), lambda i: (i, 0)),
            pl.BlockSpec((BN, SDIM), lambda i: (i, 0)),
            pl.BlockSpec((D, D), lambda i: (0, 0)),
            pl.BlockSpec((1, D), lambda i: (0, 0)),
            pl.BlockSpec((D, HID), lambda i: (0, 0)),
            pl.BlockSpec((1, HID), lambda i: (0, 0)),
            pl.BlockSpec((1, D), lambda i: (0, 0)),
            pl.BlockSpec((1, D), lambda i: (0, 0)),
        ],
        out_specs=[pl.BlockSpec((BN, GW), lambda i: (i, 0))]
        + [pl.BlockSpec((BN, CG), lambda i: (i, 0))] * NG,
        out_shape=[jax.ShapeDtypeStruct((N, GW), jnp.int32)]
        + [jax.ShapeDtypeStruct((N, CG), f32)] * NG,
    )(x_scalar, x_spherical, W1, b1.reshape(1, D), W2, b2.reshape(1, HID),
      ln_g.reshape(1, D), ln_b.reshape(1, D))

    mesh = plsc.VectorSubcoreMesh(core_axis_name="c", subcore_axis_name="s")

    gathered = pl.kernel(
        _make_gather_body(nch2, tw2),
        mesh=mesh,
        out_type=jax.ShapeDtypeStruct((E, GW), jnp.int32),
        scratch_types=[
            pltpu.VMEM((tw2, CH2), jnp.int32),
            pltpu.VMEM((CH2, GW), jnp.int32),
            pltpu.VMEM((CH2, GW), jnp.int32),
            pltpu.SemaphoreType.DMA,
            pltpu.SemaphoreType.DMA,
            pltpu.SemaphoreType.DMA,
            pltpu.SemaphoreType.DMA,
        ],
    )(g_table, src2d)

    m0, m1, m2, m3_, m4 = pl.pallas_call(
        _k3_body,
        grid=(E // BE,),
        in_specs=[
            pl.BlockSpec((BE, GW), lambda i: (i, 0)),
            pl.BlockSpec((BE, NB), lambda i: (i, 0)),
            pl.BlockSpec((BE, 1), lambda i: (i, 0)),
            pl.BlockSpec((BE, SDIM), lambda i: (i, 0)),
            pl.BlockSpec((NB, HID), lambda i: (0, 0)),
            pl.BlockSpec((1, HID), lambda i: (0, 0)),
        ],
        out_specs=[pl.BlockSpec((BE, CG), lambda i: (i, 0))] * NG,
        out_shape=[jax.ShapeDtypeStruct((E, CG), f32)] * NG,
    )(gathered, rbf, fcut, rsh, Wr, br.reshape(1, HID))

    o0, o1, o2, o3, o4 = pl.kernel(
        _make_scatter_body(nch4, tw4),
        mesh=mesh,
        out_type=[jax.ShapeDtypeStruct((N, CG), f32)] * NG,
        scratch_types=[
            pltpu.VMEM((tw4, CH), jnp.int32),
            pltpu.VMEM((CH, CG), f32),
            pltpu.VMEM((CH, CG), f32),
            pltpu.VMEM_SHARED((N + NTRASH, CG), f32),
            pltpu.SemaphoreType.DMA,
            pltpu.SemaphoreType.DMA,
            pltpu.SemaphoreType.DMA,
            pltpu.SemaphoreType.DMA,
        ],
    )(m0, m1, m2, m3_, m4, i0, i1, i2, i3, i4, dst2d)

    new_sph, new_sc = pl.pallas_call(
        _k5_body,
        grid=(nblk,),
        in_specs=[pl.BlockSpec((BN, CG), lambda i: (i, 0))] * NG,
        out_specs=[pl.BlockSpec((BN, SDIM), lambda i: (i, 0)),
                   pl.BlockSpec((BN, D), lambda i: (i, 0))],
        out_shape=[jax.ShapeDtypeStruct((N, SDIM), f32),
                   jax.ShapeDtypeStruct((N, D), f32)],
    )(o0, o1, o2, o3, o4)
    return (new_sc, new_sph)
